# Initial kernel scaffold; baseline (speedup 1.0000x reference)
import jax
import jax.numpy as jnp
from jax.experimental import pallas as pl


def kernel(x, edge_index, W1, b1, W2, b2, gamma):
    raise NotImplementedError("write your pallas kernel here")


# speedup vs baseline: 2.2094x; 2.2094x over previous
import functools

import jax
import jax.numpy as jnp
from jax import lax
from jax.experimental import pallas as pl
from jax.experimental.pallas import tpu as pltpu
from jax.experimental.pallas import tpu_sc as plsc

NS = 16
CHUNK = 128


def _dense_body(x_ref, w1_ref, b1_ref, w2_ref, b2_ref, o_ref):
    h = jnp.dot(x_ref[...], w1_ref[...], preferred_element_type=jnp.float32)
    h = jnp.maximum(h + b1_ref[...], 0.0)
    o_ref[...] = jnp.dot(h, w2_ref[...], preferred_element_type=jnp.float32) + b2_ref[...]


def _dense(x, W1, b1, W2, b2):
    n, d = x.shape
    bn = 1000
    assert n % bn == 0
    return pl.pallas_call(
        _dense_body,
        grid=(n // bn,),
        in_specs=[
            pl.BlockSpec((bn, d), lambda i: (i, 0)),
            pl.BlockSpec((d, d), lambda i: (0, 0)),
            pl.BlockSpec((1, d), lambda i: (0, 0)),
            pl.BlockSpec((d, d), lambda i: (0, 0)),
            pl.BlockSpec((1, d), lambda i: (0, 0)),
        ],
        out_specs=pl.BlockSpec((bn, d), lambda i: (i, 0)),
        out_shape=jax.ShapeDtypeStruct((n, d), jnp.float32),
    )(x, W1, b1.reshape(1, d), W2, b2.reshape(1, d))


def _prep_body(dp_ref, h_ref, g0_ref, dinv_ref, g_ref, out_ref):
    deg = jnp.sum(dp_ref[...], axis=1, keepdims=True) * (1.0 / 128.0)
    dinv = jnp.where(deg > 0, lax.rsqrt(deg), 0.0)
    dinv_ref[...] = dinv
    h = h_ref[...]
    g_ref[...] = dinv * h
    out_ref[...] = g0_ref[0, 0] * h


def _prep(degp, h_pad, gamma0):
    n_pad, d = h_pad.shape
    bn = n_pad // 10
    return pl.pallas_call(
        _prep_body,
        grid=(n_pad // bn,),
        in_specs=[
            pl.BlockSpec((bn, d), lambda i: (i, 0)),
            pl.BlockSpec((bn, d), lambda i: (i, 0)),
            pl.BlockSpec((1, 1), lambda i: (0, 0)),
        ],
        out_specs=[
            pl.BlockSpec((bn, 1), lambda i: (i, 0)),
            pl.BlockSpec((bn, d), lambda i: (i, 0)),
            pl.BlockSpec((bn, d), lambda i: (i, 0)),
        ],
        out_shape=[
            jax.ShapeDtypeStruct((n_pad, 1), jnp.float32),
            jax.ShapeDtypeStruct((n_pad, d), jnp.float32),
            jax.ShapeDtypeStruct((n_pad, d), jnp.float32),
        ],
    )(degp, h_pad, gamma0)


def _combine_body(s_ref, out_ref, dinv_ref, gk_ref, outn_ref, g_ref):
    s = s_ref[...]
    dinv = dinv_ref[...]
    hk = dinv * s
    outn_ref[...] = out_ref[...] + gk_ref[0, 0] * hk
    g_ref[...] = dinv * hk


def _combine(s, out_prev, dinv, gk):
    n_pad, d = s.shape
    bn = n_pad // 10
    return pl.pallas_call(
        _combine_body,
        grid=(n_pad // bn,),
        in_specs=[
            pl.BlockSpec((bn, d), lambda i: (i, 0)),
            pl.BlockSpec((bn, d), lambda i: (i, 0)),
            pl.BlockSpec((bn, 1), lambda i: (i, 0)),
            pl.BlockSpec((1, 1), lambda i: (0, 0)),
        ],
        out_specs=[
            pl.BlockSpec((bn, d), lambda i: (i, 0)),
            pl.BlockSpec((bn, d), lambda i: (i, 0)),
        ],
        out_shape=[
            jax.ShapeDtypeStruct((n_pad, d), jnp.float32),
            jax.ShapeDtypeStruct((n_pad, d), jnp.float32),
        ],
    )(s, out_prev, dinv, gk)


def _sc_mesh():
    return plsc.VectorSubcoreMesh(
        core_axis_name="c", subcore_axis_name="s", num_cores=1)


def _sc_degree(col2d, ones_blk, zeros128, n_pad):
    c_pad = col2d.shape[0]
    cpt = c_pad // NS
    rpt = n_pad // NS
    d = ones_blk.shape[1]

    @functools.partial(
        pl.kernel,
        out_type=jax.ShapeDtypeStruct((n_pad, d), jnp.float32),
        mesh=_sc_mesh(),
        scratch_types=[
            pltpu.VMEM((CHUNK,), jnp.int32),
            pltpu.VMEM((CHUNK, d), jnp.float32),
            pltpu.VMEM_SHARED((n_pad, d), jnp.float32),
        ],
    )
    def deg_kernel(col_hbm, ones_hbm, zeros_hbm, out, cidx_v, ones_v, acc):
        sid = lax.axis_index("s")
        r0 = sid * rpt
        pltpu.sync_copy(zeros_hbm.at[pl.ds(r0, rpt)], acc.at[pl.ds(r0, rpt)])
        pltpu.sync_copy(ones_hbm, ones_v)
        plsc.subcore_barrier()

        def body(j, carry):
            pltpu.sync_copy(col_hbm.at[sid * cpt + j], cidx_v)
            pltpu.sync_copy(ones_v, acc.at[cidx_v], add=True)
            return carry

        lax.fori_loop(0, cpt, body, 0)
        plsc.subcore_barrier()
        pltpu.sync_copy(acc.at[pl.ds(r0, rpt)], out.at[pl.ds(r0, rpt)])

    return deg_kernel(col2d, ones_blk, zeros128)


def _sc_propagate(g_tab, row2d, col2d, zeros128, n_pad):
    c_pad = row2d.shape[0]
    cpt = c_pad // NS
    rpt = n_pad // NS
    d = g_tab.shape[1]

    @functools.partial(
        pl.kernel,
        out_type=jax.ShapeDtypeStruct((n_pad, d), jnp.float32),
        mesh=_sc_mesh(),
        scratch_types=[
            pltpu.VMEM((CHUNK,), jnp.int32),
            pltpu.VMEM((CHUNK,), jnp.int32),
            pltpu.VMEM((CHUNK, d), jnp.float32),
            pltpu.VMEM_SHARED((n_pad, d), jnp.float32),
            pltpu.SemaphoreType.DMA,
        ],
    )
    def prop_kernel(g_hbm, row_hbm, col_hbm, zeros_hbm, out,
                    ridx_v, cidx_v, rows_v, acc, sem):
        sid = lax.axis_index("s")
        r0 = sid * rpt
        pltpu.sync_copy(zeros_hbm.at[pl.ds(r0, rpt)], acc.at[pl.ds(r0, rpt)])
        plsc.subcore_barrier()

        def body(j, carry):
            c = sid * cpt + j
            pltpu.sync_copy(row_hbm.at[c], ridx_v)
            pltpu.sync_copy(col_hbm.at[c], cidx_v)
            pltpu.async_copy(g_hbm.at[ridx_v], rows_v, sem).wait()
            pltpu.sync_copy(rows_v, acc.at[cidx_v], add=True)
            return carry

        lax.fori_loop(0, cpt, body, 0)
        plsc.subcore_barrier()
        pltpu.sync_copy(acc.at[pl.ds(r0, rpt)], out.at[pl.ds(r0, rpt)])

    return prop_kernel(g_tab, row2d, col2d, zeros128)


def kernel(x, edge_index, W1, b1, W2, b2, gamma):
    n, d = x.shape
    e = edge_index.shape[1]
    k_steps = gamma.shape[0] - 1

    n_pad = ((n + 2047) // 2048) * 2048
    if n_pad == n:
        n_pad += 2048
    e_unit = NS * 8 * CHUNK
    e_pad = ((e + e_unit - 1) // e_unit) * e_unit

    row = edge_index[0]
    col = edge_index[1]
    if e_pad != e:
        pad = e_pad - e
        row = jnp.concatenate([row, jnp.zeros((pad,), jnp.int32)])
        col = jnp.concatenate([col, jnp.full((pad,), n_pad - 1, jnp.int32)])
    row2d = row.reshape(e_pad // CHUNK, CHUNK)
    col2d = col.reshape(e_pad // CHUNK, CHUNK)

    zeros128 = jnp.zeros((n_pad, d), jnp.float32)
    ones_blk = jnp.ones((CHUNK, d), jnp.float32)

    h = _dense(x, W1, b1, W2, b2)
    h_pad = jnp.pad(h, ((0, n_pad - n), (0, 0)))

    degp = _sc_degree(col2d, ones_blk, zeros128, n_pad)
    dinv, g, out = _prep(degp, h_pad, gamma[0].reshape(1, 1))

    for k in range(1, k_steps + 1):
        s = _sc_propagate(g, row2d, col2d, zeros128, n_pad)
        out, g = _combine(s, out, dinv, gamma[k].reshape(1, 1))

    return out[:n]


# speedup vs baseline: 3.0096x; 1.3622x over previous
import functools

import jax
import jax.numpy as jnp
from jax import lax
from jax.experimental import pallas as pl
from jax.experimental.pallas import tpu as pltpu
from jax.experimental.pallas import tpu_sc as plsc

NS = 16
CHUNK = 128


def _dense_body(x_ref, w1_ref, b1_ref, w2_ref, b2_ref, o_ref):
    h = jnp.dot(x_ref[...], w1_ref[...], preferred_element_type=jnp.float32)
    h = jnp.maximum(h + b1_ref[...], 0.0)
    o_ref[...] = jnp.dot(h, w2_ref[...], preferred_element_type=jnp.float32) + b2_ref[...]


def _dense(x, W1, b1, W2, b2):
    n, d = x.shape
    bn = 1000
    assert n % bn == 0
    return pl.pallas_call(
        _dense_body,
        grid=(n // bn,),
        in_specs=[
            pl.BlockSpec((bn, d), lambda i: (i, 0)),
            pl.BlockSpec((d, d), lambda i: (0, 0)),
            pl.BlockSpec((1, d), lambda i: (0, 0)),
            pl.BlockSpec((d, d), lambda i: (0, 0)),
            pl.BlockSpec((1, d), lambda i: (0, 0)),
        ],
        out_specs=pl.BlockSpec((bn, d), lambda i: (i, 0)),
        out_shape=jax.ShapeDtypeStruct((n, d), jnp.float32),
    )(x, W1, b1.reshape(1, d), W2, b2.reshape(1, d))


def _prep_body(dp_ref, h_ref, g0_ref, dinv_ref, g_ref, out_ref):
    deg = jnp.sum(dp_ref[...], axis=1, keepdims=True) * (1.0 / 128.0)
    dinv = jnp.where(deg > 0, lax.rsqrt(deg), 0.0)
    dinv_ref[...] = dinv
    h = h_ref[...]
    g_ref[...] = dinv * h
    out_ref[...] = g0_ref[0, 0] * h


def _prep(degp, h_pad, gamma0):
    n_pad, d = h_pad.shape
    bn = n_pad // 10
    return pl.pallas_call(
        _prep_body,
        grid=(n_pad // bn,),
        in_specs=[
            pl.BlockSpec((bn, d), lambda i: (i, 0)),
            pl.BlockSpec((bn, d), lambda i: (i, 0)),
            pl.BlockSpec((1, 1), lambda i: (0, 0)),
        ],
        out_specs=[
            pl.BlockSpec((bn, 1), lambda i: (i, 0)),
            pl.BlockSpec((bn, d), lambda i: (i, 0)),
            pl.BlockSpec((bn, d), lambda i: (i, 0)),
        ],
        out_shape=[
            jax.ShapeDtypeStruct((n_pad, 1), jnp.float32),
            jax.ShapeDtypeStruct((n_pad, d), jnp.float32),
            jax.ShapeDtypeStruct((n_pad, d), jnp.float32),
        ],
    )(degp, h_pad, gamma0)


def _combine_body(s_ref, out_ref, dinv_ref, gk_ref, outn_ref, g_ref):
    s = s_ref[...]
    dinv = dinv_ref[...]
    hk = dinv * s
    outn_ref[...] = out_ref[...] + gk_ref[0, 0] * hk
    g_ref[...] = dinv * hk


def _combine(s, out_prev, dinv, gk):
    n_pad, d = s.shape
    bn = n_pad // 10
    return pl.pallas_call(
        _combine_body,
        grid=(n_pad // bn,),
        in_specs=[
            pl.BlockSpec((bn, d), lambda i: (i, 0)),
            pl.BlockSpec((bn, d), lambda i: (i, 0)),
            pl.BlockSpec((bn, 1), lambda i: (i, 0)),
            pl.BlockSpec((1, 1), lambda i: (0, 0)),
        ],
        out_specs=[
            pl.BlockSpec((bn, d), lambda i: (i, 0)),
            pl.BlockSpec((bn, d), lambda i: (i, 0)),
        ],
        out_shape=[
            jax.ShapeDtypeStruct((n_pad, d), jnp.float32),
            jax.ShapeDtypeStruct((n_pad, d), jnp.float32),
        ],
    )(s, out_prev, dinv, gk)


def _sc_mesh():
    return plsc.VectorSubcoreMesh(
        core_axis_name="c", subcore_axis_name="s", num_cores=1)


def _sc_degree(col2d, ones_blk, zeros128, n_pad):
    c_pad = col2d.shape[0]
    cpt = c_pad // NS
    rpt = n_pad // NS
    d = ones_blk.shape[1]

    @functools.partial(
        pl.kernel,
        out_type=jax.ShapeDtypeStruct((n_pad, d), jnp.float32),
        mesh=_sc_mesh(),
        scratch_types=[
            pltpu.VMEM((CHUNK, d), jnp.float32),
            pltpu.VMEM((CHUNK,), jnp.int32),
            pltpu.VMEM((CHUNK,), jnp.int32),
            pltpu.VMEM((CHUNK,), jnp.int32),
            pltpu.VMEM((CHUNK,), jnp.int32),
            pltpu.VMEM_SHARED((n_pad, d), jnp.float32),
            pltpu.SemaphoreType.DMA,
            pltpu.SemaphoreType.DMA,
            pltpu.SemaphoreType.DMA,
            pltpu.SemaphoreType.DMA,
            pltpu.SemaphoreType.DMA,
            pltpu.SemaphoreType.DMA,
        ],
    )
    def deg_kernel(col_hbm, ones_hbm, zeros_hbm, out, ones_v,
                   ci0, ci1, ci2, ci3, acc, cs0, cs1, cs2, cs3, ss0, ss1):
        cidx = (ci0, ci1, ci2, ci3)
        cis = (cs0, cs1, cs2, cs3)
        ss = (ss0, ss1)
        sid = lax.axis_index("s")
        r0 = sid * rpt
        base = sid * cpt
        pltpu.sync_copy(zeros_hbm.at[pl.ds(r0, rpt)], acc.at[pl.ds(r0, rpt)])
        pltpu.sync_copy(ones_hbm, ones_v)
        pltpu.sync_copy(col_hbm.at[base], cidx[0])
        plsc.subcore_barrier()

        def outer(j4, carry):
            for b in range(4):
                j = j4 * 4 + b
                b2 = b % 2

                @pl.when(j >= 2)
                def _():
                    pltpu.make_async_copy(
                        ones_v, acc.at[cidx[(b - 2) % 4]], ss[b2]).wait()

                @pl.when(j + 1 < cpt)
                def _():
                    pltpu.async_copy(col_hbm.at[base + j + 1],
                                     cidx[(b + 1) % 4], cis[(b + 1) % 4])

                @pl.when(j >= 1)
                def _():
                    pltpu.make_async_copy(col_hbm.at[base + j],
                                          cidx[b], cis[b]).wait()

                pltpu.async_copy(ones_v, acc.at[cidx[b]], ss[b2], add=True)
            return carry

        lax.fori_loop(0, cpt // 4, outer, 0)
        pltpu.make_async_copy(ones_v, acc.at[cidx[(cpt - 2) % 4]], ss[0]).wait()
        pltpu.make_async_copy(ones_v, acc.at[cidx[(cpt - 1) % 4]], ss[1]).wait()
        plsc.subcore_barrier()
        pltpu.sync_copy(acc.at[pl.ds(r0, rpt)], out.at[pl.ds(r0, rpt)])

    return deg_kernel(col2d, ones_blk, zeros128)


def _sc_propagate(g_tab, row2d, col2d, zeros128, n_pad):
    c_pad = row2d.shape[0]
    cpt = c_pad // NS
    rpt = n_pad // NS
    d = g_tab.shape[1]

    @functools.partial(
        pl.kernel,
        out_type=jax.ShapeDtypeStruct((n_pad, d), jnp.float32),
        mesh=_sc_mesh(),
        scratch_types=[
            pltpu.VMEM((CHUNK,), jnp.int32),
            pltpu.VMEM((CHUNK,), jnp.int32),
            pltpu.VMEM((CHUNK,), jnp.int32),
            pltpu.VMEM((CHUNK,), jnp.int32),
            pltpu.VMEM((CHUNK,), jnp.int32),
            pltpu.VMEM((CHUNK,), jnp.int32),
            pltpu.VMEM((CHUNK,), jnp.int32),
            pltpu.VMEM((CHUNK,), jnp.int32),
            pltpu.VMEM((CHUNK, d), jnp.float32),
            pltpu.VMEM((CHUNK, d), jnp.float32),
            pltpu.VMEM_SHARED((n_pad, d), jnp.float32),
            pltpu.SemaphoreType.DMA,
            pltpu.SemaphoreType.DMA,
            pltpu.SemaphoreType.DMA,
            pltpu.SemaphoreType.DMA,
            pltpu.SemaphoreType.DMA,
            pltpu.SemaphoreType.DMA,
            pltpu.SemaphoreType.DMA,
            pltpu.SemaphoreType.DMA,
            pltpu.SemaphoreType.DMA,
            pltpu.SemaphoreType.DMA,
            pltpu.SemaphoreType.DMA,
            pltpu.SemaphoreType.DMA,
        ],
    )
    def prop_kernel(g_hbm, row_hbm, col_hbm, zeros_hbm, out,
                    ri0, ri1, ri2, ri3, ci0, ci1, ci2, ci3, rows0, rows1, acc,
                    rs0, rs1, rs2, rs3, cs0, cs1, cs2, cs3, gs0, gs1, ss0, ss1):
        ridx = (ri0, ri1, ri2, ri3)
        cidx = (ci0, ci1, ci2, ci3)
        rows = (rows0, rows1)
        ris = (rs0, rs1, rs2, rs3)
        cis = (cs0, cs1, cs2, cs3)
        gs = (gs0, gs1)
        ss = (ss0, ss1)
        sid = lax.axis_index("s")
        r0 = sid * rpt
        base = sid * cpt
        pltpu.sync_copy(zeros_hbm.at[pl.ds(r0, rpt)], acc.at[pl.ds(r0, rpt)])
        pltpu.sync_copy(row_hbm.at[base], ridx[0])
        pltpu.sync_copy(col_hbm.at[base], cidx[0])
        plsc.subcore_barrier()

        def outer(j4, carry):
            for b in range(4):
                j = j4 * 4 + b
                b2 = b % 2

                @pl.when(j >= 2)
                def _():
                    pltpu.make_async_copy(
                        rows[b2], acc.at[cidx[(b - 2) % 4]], ss[b2]).wait()

                @pl.when(j >= 1)
                def _():
                    pltpu.make_async_copy(row_hbm.at[base + j],
                                          ridx[b], ris[b]).wait()

                pltpu.async_copy(g_hbm.at[ridx[b]], rows[b2], gs[b2])

                @pl.when(j + 1 < cpt)
                def _():
                    pltpu.async_copy(row_hbm.at[base + j + 1],
                                     ridx[(b + 1) % 4], ris[(b + 1) % 4])
                    pltpu.async_copy(col_hbm.at[base + j + 1],
                                     cidx[(b + 1) % 4], cis[(b + 1) % 4])

                pltpu.make_async_copy(g_hbm.at[ridx[b]], rows[b2], gs[b2]).wait()

                @pl.when(j >= 1)
                def _():
                    pltpu.make_async_copy(col_hbm.at[base + j],
                                          cidx[b], cis[b]).wait()

                pltpu.async_copy(rows[b2], acc.at[cidx[b]], ss[b2], add=True)
            return carry

        lax.fori_loop(0, cpt // 4, outer, 0)
        pltpu.make_async_copy(rows[0], acc.at[cidx[(cpt - 2) % 4]], ss[0]).wait()
        pltpu.make_async_copy(rows[1], acc.at[cidx[(cpt - 1) % 4]], ss[1]).wait()
        plsc.subcore_barrier()
        pltpu.sync_copy(acc.at[pl.ds(r0, rpt)], out.at[pl.ds(r0, rpt)])

    return prop_kernel(g_tab, row2d, col2d, zeros128)


def kernel(x, edge_index, W1, b1, W2, b2, gamma):
    n, d = x.shape
    e = edge_index.shape[1]
    k_steps = gamma.shape[0] - 1

    n_pad = ((n + 2047) // 2048) * 2048
    if n_pad == n:
        n_pad += 2048
    e_unit = NS * 8 * CHUNK
    e_pad = ((e + e_unit - 1) // e_unit) * e_unit

    row = edge_index[0]
    col = edge_index[1]
    if e_pad != e:
        pad = e_pad - e
        row = jnp.concatenate([row, jnp.zeros((pad,), jnp.int32)])
        col = jnp.concatenate([col, jnp.full((pad,), n_pad - 1, jnp.int32)])
    row2d = row.reshape(e_pad // CHUNK, CHUNK)
    col2d = col.reshape(e_pad // CHUNK, CHUNK)

    zeros128 = jnp.zeros((n_pad, d), jnp.float32)
    ones_blk = jnp.ones((CHUNK, d), jnp.float32)

    h = _dense(x, W1, b1, W2, b2)
    h_pad = jnp.pad(h, ((0, n_pad - n), (0, 0)))

    degp = _sc_degree(col2d, ones_blk, zeros128, n_pad)
    dinv, g, out = _prep(degp, h_pad, gamma[0].reshape(1, 1))

    for k in range(1, k_steps + 1):
        s = _sc_propagate(g, row2d, col2d, zeros128, n_pad)
        out, g = _combine(s, out, dinv, gamma[k].reshape(1, 1))

    return out[:n]


# speedup vs baseline: 3.0587x; 1.0163x over previous
import functools

import jax
import jax.numpy as jnp
from jax import lax
from jax.experimental import pallas as pl
from jax.experimental.pallas import tpu as pltpu
from jax.experimental.pallas import tpu_sc as plsc

NS = 16
CHUNK = 128


def _dense_body(x_ref, w1_ref, b1_ref, w2_ref, b2_ref, o_ref):
    h = jnp.dot(x_ref[...], w1_ref[...], preferred_element_type=jnp.float32)
    h = jnp.maximum(h + b1_ref[...], 0.0)
    o_ref[...] = jnp.dot(h, w2_ref[...], preferred_element_type=jnp.float32) + b2_ref[...]


def _dense(x, W1, b1, W2, b2):
    n, d = x.shape
    bn = 1000
    assert n % bn == 0
    return pl.pallas_call(
        _dense_body,
        grid=(n // bn,),
        in_specs=[
            pl.BlockSpec((bn, d), lambda i: (i, 0)),
            pl.BlockSpec((d, d), lambda i: (0, 0)),
            pl.BlockSpec((1, d), lambda i: (0, 0)),
            pl.BlockSpec((d, d), lambda i: (0, 0)),
            pl.BlockSpec((1, d), lambda i: (0, 0)),
        ],
        out_specs=pl.BlockSpec((bn, d), lambda i: (i, 0)),
        out_shape=jax.ShapeDtypeStruct((n, d), jnp.float32),
    )(x, W1, b1.reshape(1, d), W2, b2.reshape(1, d))


def _prep_body(dp_ref, h_ref, g0_ref, dinv_ref, g_ref, out_ref):
    deg = jnp.sum(dp_ref[...], axis=1, keepdims=True) * (1.0 / 128.0)
    dinv = jnp.where(deg > 0, lax.rsqrt(deg), 0.0)
    dinv_ref[...] = dinv
    h = h_ref[...]
    g_ref[...] = dinv * h
    out_ref[...] = g0_ref[0, 0] * h


def _prep(degp, h_pad, gamma0):
    n_pad, d = h_pad.shape
    bn = n_pad // 10
    return pl.pallas_call(
        _prep_body,
        grid=(n_pad // bn,),
        in_specs=[
            pl.BlockSpec((bn, d), lambda i: (i, 0)),
            pl.BlockSpec((bn, d), lambda i: (i, 0)),
            pl.BlockSpec((1, 1), lambda i: (0, 0)),
        ],
        out_specs=[
            pl.BlockSpec((bn, 1), lambda i: (i, 0)),
            pl.BlockSpec((bn, d), lambda i: (i, 0)),
            pl.BlockSpec((bn, d), lambda i: (i, 0)),
        ],
        out_shape=[
            jax.ShapeDtypeStruct((n_pad, 1), jnp.float32),
            jax.ShapeDtypeStruct((n_pad, d), jnp.float32),
            jax.ShapeDtypeStruct((n_pad, d), jnp.float32),
        ],
    )(degp, h_pad, gamma0)


def _combine_body(p0_ref, p1_ref, out_ref, dinv_ref, gk_ref, outn_ref, g_ref):
    s = p0_ref[...] + p1_ref[...]
    dinv = dinv_ref[...]
    hk = dinv * s
    outn_ref[...] = out_ref[...] + gk_ref[0, 0] * hk
    g_ref[...] = dinv * hk


def _combine(p0, p1, out_prev, dinv, gk):
    n_pad, d = p0.shape
    bn = n_pad // 10
    return pl.pallas_call(
        _combine_body,
        grid=(n_pad // bn,),
        in_specs=[
            pl.BlockSpec((bn, d), lambda i: (i, 0)),
            pl.BlockSpec((bn, d), lambda i: (i, 0)),
            pl.BlockSpec((bn, d), lambda i: (i, 0)),
            pl.BlockSpec((bn, 1), lambda i: (i, 0)),
            pl.BlockSpec((1, 1), lambda i: (0, 0)),
        ],
        out_specs=[
            pl.BlockSpec((bn, d), lambda i: (i, 0)),
            pl.BlockSpec((bn, d), lambda i: (i, 0)),
        ],
        out_shape=[
            jax.ShapeDtypeStruct((n_pad, d), jnp.float32),
            jax.ShapeDtypeStruct((n_pad, d), jnp.float32),
        ],
    )(p0, p1, out_prev, dinv, gk)


def _sc_mesh():
    return plsc.VectorSubcoreMesh(
        core_axis_name="c", subcore_axis_name="s", num_cores=1)


def _sc_degree(col2d, ones_blk, zeros128, n_pad):
    c_pad = col2d.shape[0]
    cpt = c_pad // NS
    rpt = n_pad // NS
    d = ones_blk.shape[1]

    @functools.partial(
        pl.kernel,
        out_type=jax.ShapeDtypeStruct((n_pad, d), jnp.float32),
        mesh=_sc_mesh(),
        scratch_types=[
            pltpu.VMEM((CHUNK, d), jnp.float32),
            pltpu.VMEM((CHUNK,), jnp.int32),
            pltpu.VMEM((CHUNK,), jnp.int32),
            pltpu.VMEM((CHUNK,), jnp.int32),
            pltpu.VMEM((CHUNK,), jnp.int32),
            pltpu.VMEM_SHARED((n_pad, d), jnp.float32),
            pltpu.SemaphoreType.DMA,
            pltpu.SemaphoreType.DMA,
            pltpu.SemaphoreType.DMA,
            pltpu.SemaphoreType.DMA,
            pltpu.SemaphoreType.DMA,
            pltpu.SemaphoreType.DMA,
        ],
    )
    def deg_kernel(col_hbm, ones_hbm, zeros_hbm, out, ones_v,
                   ci0, ci1, ci2, ci3, acc, cs0, cs1, cs2, cs3, ss0, ss1):
        cidx = (ci0, ci1, ci2, ci3)
        cis = (cs0, cs1, cs2, cs3)
        ss = (ss0, ss1)
        sid = lax.axis_index("s")
        r0 = sid * rpt
        base = sid * cpt
        pltpu.sync_copy(zeros_hbm.at[pl.ds(r0, rpt)], acc.at[pl.ds(r0, rpt)])
        pltpu.sync_copy(ones_hbm, ones_v)
        pltpu.sync_copy(col_hbm.at[base], cidx[0])
        plsc.subcore_barrier()

        def outer(j4, carry):
            for b in range(4):
                j = j4 * 4 + b
                b2 = b % 2

                @pl.when(j >= 2)
                def _():
                    pltpu.make_async_copy(
                        ones_v, acc.at[cidx[(b - 2) % 4]], ss[b2]).wait()

                @pl.when(j + 1 < cpt)
                def _():
                    pltpu.async_copy(col_hbm.at[base + j + 1],
                                     cidx[(b + 1) % 4], cis[(b + 1) % 4])

                @pl.when(j >= 1)
                def _():
                    pltpu.make_async_copy(col_hbm.at[base + j],
                                          cidx[b], cis[b]).wait()

                pltpu.async_copy(ones_v, acc.at[cidx[b]], ss[b2], add=True)
            return carry

        lax.fori_loop(0, cpt // 4, outer, 0)
        pltpu.make_async_copy(ones_v, acc.at[cidx[(cpt - 2) % 4]], ss[0]).wait()
        pltpu.make_async_copy(ones_v, acc.at[cidx[(cpt - 1) % 4]], ss[1]).wait()
        plsc.subcore_barrier()
        pltpu.sync_copy(acc.at[pl.ds(r0, rpt)], out.at[pl.ds(r0, rpt)])

    return deg_kernel(col2d, ones_blk, zeros128)


def _sc_propagate(g_tab, row2d, col2d, zeros128, n_pad):
    c_pad = row2d.shape[0]
    cpt = c_pad // NS
    rpt = n_pad // NS
    d = g_tab.shape[1]

    @functools.partial(
        pl.kernel,
        out_type=jax.ShapeDtypeStruct((n_pad, d), jnp.float32),
        mesh=_sc_mesh(),
        scratch_types=[
            pltpu.VMEM((CHUNK,), jnp.int32),
            pltpu.VMEM((CHUNK,), jnp.int32),
            pltpu.VMEM((CHUNK,), jnp.int32),
            pltpu.VMEM((CHUNK,), jnp.int32),
            pltpu.VMEM((CHUNK,), jnp.int32),
            pltpu.VMEM((CHUNK,), jnp.int32),
            pltpu.VMEM((CHUNK,), jnp.int32),
            pltpu.VMEM((CHUNK,), jnp.int32),
            pltpu.VMEM((CHUNK, d), jnp.float32),
            pltpu.VMEM((CHUNK, d), jnp.float32),
            pltpu.VMEM_SHARED((n_pad, d), jnp.float32),
            pltpu.SemaphoreType.DMA,
            pltpu.SemaphoreType.DMA,
            pltpu.SemaphoreType.DMA,
            pltpu.SemaphoreType.DMA,
            pltpu.SemaphoreType.DMA,
            pltpu.SemaphoreType.DMA,
            pltpu.SemaphoreType.DMA,
            pltpu.SemaphoreType.DMA,
            pltpu.SemaphoreType.DMA,
            pltpu.SemaphoreType.DMA,
            pltpu.SemaphoreType.DMA,
            pltpu.SemaphoreType.DMA,
        ],
    )
    def prop_kernel(g_hbm, row_hbm, col_hbm, zeros_hbm, out,
                    ri0, ri1, ri2, ri3, ci0, ci1, ci2, ci3, rows0, rows1, acc,
                    rs0, rs1, rs2, rs3, cs0, cs1, cs2, cs3, gs0, gs1, ss0, ss1):
        ridx = (ri0, ri1, ri2, ri3)
        cidx = (ci0, ci1, ci2, ci3)
        rows = (rows0, rows1)
        ris = (rs0, rs1, rs2, rs3)
        cis = (cs0, cs1, cs2, cs3)
        gs = (gs0, gs1)
        ss = (ss0, ss1)
        sid = lax.axis_index("s")
        r0 = sid * rpt
        base = sid * cpt
        pltpu.sync_copy(zeros_hbm.at[pl.ds(r0, rpt)], acc.at[pl.ds(r0, rpt)])
        pltpu.sync_copy(row_hbm.at[base], ridx[0])
        pltpu.sync_copy(col_hbm.at[base], cidx[0])
        plsc.subcore_barrier()

        def outer(j4, carry):
            for b in range(4):
                j = j4 * 4 + b
                b2 = b % 2

                @pl.when(j >= 2)
                def _():
                    pltpu.make_async_copy(
                        rows[b2], acc.at[cidx[(b - 2) % 4]], ss[b2]).wait()

                @pl.when(j >= 1)
                def _():
                    pltpu.make_async_copy(row_hbm.at[base + j],
                                          ridx[b], ris[b]).wait()

                pltpu.async_copy(g_hbm.at[ridx[b]], rows[b2], gs[b2])

                @pl.when(j + 1 < cpt)
                def _():
                    pltpu.async_copy(row_hbm.at[base + j + 1],
                                     ridx[(b + 1) % 4], ris[(b + 1) % 4])
                    pltpu.async_copy(col_hbm.at[base + j + 1],
                                     cidx[(b + 1) % 4], cis[(b + 1) % 4])

                pltpu.make_async_copy(g_hbm.at[ridx[b]], rows[b2], gs[b2]).wait()

                @pl.when(j >= 1)
                def _():
                    pltpu.make_async_copy(col_hbm.at[base + j],
                                          cidx[b], cis[b]).wait()

                pltpu.async_copy(rows[b2], acc.at[cidx[b]], ss[b2], add=True)
            return carry

        lax.fori_loop(0, cpt // 4, outer, 0)
        pltpu.make_async_copy(rows[0], acc.at[cidx[(cpt - 2) % 4]], ss[0]).wait()
        pltpu.make_async_copy(rows[1], acc.at[cidx[(cpt - 1) % 4]], ss[1]).wait()
        plsc.subcore_barrier()
        pltpu.sync_copy(acc.at[pl.ds(r0, rpt)], out.at[pl.ds(r0, rpt)])

    return prop_kernel(g_tab, row2d, col2d, zeros128)


def kernel(x, edge_index, W1, b1, W2, b2, gamma):
    n, d = x.shape
    e = edge_index.shape[1]
    k_steps = gamma.shape[0] - 1

    n_pad = ((n + 2047) // 2048) * 2048
    if n_pad == n:
        n_pad += 2048
    e_unit = NS * 8 * CHUNK
    e_pad = ((e + e_unit - 1) // e_unit) * e_unit

    row = edge_index[0]
    col = edge_index[1]
    if e_pad != e:
        pad = e_pad - e
        row = jnp.concatenate([row, jnp.zeros((pad,), jnp.int32)])
        col = jnp.concatenate([col, jnp.full((pad,), n_pad - 1, jnp.int32)])
    row2d = row.reshape(e_pad // CHUNK, CHUNK)
    col2d = col.reshape(e_pad // CHUNK, CHUNK)

    zeros128 = jnp.zeros((n_pad, d), jnp.float32)
    ones_blk = jnp.ones((CHUNK, d), jnp.float32)

    h = _dense(x, W1, b1, W2, b2)
    h_pad = jnp.pad(h, ((0, n_pad - n), (0, 0)))

    degp = _sc_degree(col2d, ones_blk, zeros128, n_pad)
    dinv, g, out = _prep(degp, h_pad, gamma[0].reshape(1, 1))

    half = row2d.shape[0] // 2
    row_a, row_b = row2d[:half], row2d[half:]
    col_a, col_b = col2d[:half], col2d[half:]
    for k in range(1, k_steps + 1):
        p0 = _sc_propagate(g, row_a, col_a, zeros128, n_pad)
        p1 = _sc_propagate(g, row_b, col_b, zeros128, n_pad)
        out, g = _combine(p0, p1, out, dinv, gamma[k].reshape(1, 1))

    return out[:n]


# speedup vs baseline: 3.3000x; 1.0789x over previous
import functools

import jax
import jax.numpy as jnp
from jax import lax
from jax.experimental import pallas as pl
from jax.experimental.pallas import tpu as pltpu
from jax.experimental.pallas import tpu_sc as plsc

NS = 16
CHUNK = 128


def _dense_body(x_ref, w1_ref, b1_ref, w2_ref, b2_ref, o_ref):
    h = jnp.dot(x_ref[...], w1_ref[...], preferred_element_type=jnp.float32)
    h = jnp.maximum(h + b1_ref[...], 0.0)
    o_ref[...] = jnp.dot(h, w2_ref[...], preferred_element_type=jnp.float32) + b2_ref[...]


def _dense(x, W1, b1, W2, b2):
    n, d = x.shape
    bn = 1000
    assert n % bn == 0
    return pl.pallas_call(
        _dense_body,
        grid=(n // bn,),
        in_specs=[
            pl.BlockSpec((bn, d), lambda i: (i, 0)),
            pl.BlockSpec((d, d), lambda i: (0, 0)),
            pl.BlockSpec((1, d), lambda i: (0, 0)),
            pl.BlockSpec((d, d), lambda i: (0, 0)),
            pl.BlockSpec((1, d), lambda i: (0, 0)),
        ],
        out_specs=pl.BlockSpec((bn, d), lambda i: (i, 0)),
        out_shape=jax.ShapeDtypeStruct((n, d), jnp.float32),
    )(x, W1, b1.reshape(1, d), W2, b2.reshape(1, d))


def _prep_body(dp_ref, h_ref, g0_ref, dinv_ref, g_ref, out_ref):
    deg = jnp.sum(dp_ref[...], axis=1, keepdims=True) * (1.0 / 128.0)
    dinv = jnp.where(deg > 0, lax.rsqrt(deg), 0.0)
    dinv_ref[...] = dinv
    h = h_ref[...]
    g_ref[...] = dinv * h
    out_ref[...] = g0_ref[0, 0] * h


def _prep(degp, h_pad, gamma0):
    n_pad, d = h_pad.shape
    bn = n_pad // 10
    return pl.pallas_call(
        _prep_body,
        grid=(n_pad // bn,),
        in_specs=[
            pl.BlockSpec((bn, d), lambda i: (i, 0)),
            pl.BlockSpec((bn, d), lambda i: (i, 0)),
            pl.BlockSpec((1, 1), lambda i: (0, 0)),
        ],
        out_specs=[
            pl.BlockSpec((bn, 1), lambda i: (i, 0)),
            pl.BlockSpec((bn, d), lambda i: (i, 0)),
            pl.BlockSpec((bn, d), lambda i: (i, 0)),
        ],
        out_shape=[
            jax.ShapeDtypeStruct((n_pad, 1), jnp.float32),
            jax.ShapeDtypeStruct((n_pad, d), jnp.float32),
            jax.ShapeDtypeStruct((n_pad, d), jnp.float32),
        ],
    )(degp, h_pad, gamma0)


def _combine_body(p0_ref, p1_ref, out_ref, dinv_ref, gk_ref, outn_ref, g_ref):
    s = p0_ref[...] + p1_ref[...]
    dinv = dinv_ref[...]
    hk = dinv * s
    outn_ref[...] = out_ref[...] + gk_ref[0, 0] * hk
    g_ref[...] = dinv * hk


def _combine(p0, p1, out_prev, dinv, gk):
    n_pad, d = p0.shape
    bn = n_pad // 10
    return pl.pallas_call(
        _combine_body,
        grid=(n_pad // bn,),
        in_specs=[
            pl.BlockSpec((bn, d), lambda i: (i, 0)),
            pl.BlockSpec((bn, d), lambda i: (i, 0)),
            pl.BlockSpec((bn, d), lambda i: (i, 0)),
            pl.BlockSpec((bn, 1), lambda i: (i, 0)),
            pl.BlockSpec((1, 1), lambda i: (0, 0)),
        ],
        out_specs=[
            pl.BlockSpec((bn, d), lambda i: (i, 0)),
            pl.BlockSpec((bn, d), lambda i: (i, 0)),
        ],
        out_shape=[
            jax.ShapeDtypeStruct((n_pad, d), jnp.float32),
            jax.ShapeDtypeStruct((n_pad, d), jnp.float32),
        ],
    )(p0, p1, out_prev, dinv, gk)


def _sc_mesh():
    return plsc.VectorSubcoreMesh(
        core_axis_name="c", subcore_axis_name="s", num_cores=1)


def _sc_degree(col2d, ones_blk, zeros128, n_pad):
    c_pad = col2d.shape[0]
    cpt = c_pad // NS
    rpt = n_pad // NS
    d = ones_blk.shape[1]

    @functools.partial(
        pl.kernel,
        out_type=jax.ShapeDtypeStruct((n_pad, d), jnp.float32),
        mesh=_sc_mesh(),
        scratch_types=[
            pltpu.VMEM((CHUNK, d), jnp.float32),
            pltpu.VMEM((CHUNK,), jnp.int32),
            pltpu.VMEM((CHUNK,), jnp.int32),
            pltpu.VMEM((CHUNK,), jnp.int32),
            pltpu.VMEM((CHUNK,), jnp.int32),
            pltpu.VMEM_SHARED((n_pad, d), jnp.float32),
            pltpu.SemaphoreType.DMA,
            pltpu.SemaphoreType.DMA,
            pltpu.SemaphoreType.DMA,
            pltpu.SemaphoreType.DMA,
            pltpu.SemaphoreType.DMA,
            pltpu.SemaphoreType.DMA,
        ],
    )
    def deg_kernel(col_hbm, ones_hbm, zeros_hbm, out, ones_v,
                   ci0, ci1, ci2, ci3, acc, cs0, cs1, cs2, cs3, ss0, ss1):
        cidx = (ci0, ci1, ci2, ci3)
        cis = (cs0, cs1, cs2, cs3)
        ss = (ss0, ss1)
        sid = lax.axis_index("s")
        r0 = sid * rpt
        base = sid * cpt
        pltpu.sync_copy(zeros_hbm.at[pl.ds(r0, rpt)], acc.at[pl.ds(r0, rpt)])
        pltpu.sync_copy(ones_hbm, ones_v)
        pltpu.sync_copy(col_hbm.at[base], cidx[0])
        plsc.subcore_barrier()

        def outer(j4, carry):
            for b in range(4):
                j = j4 * 4 + b
                b2 = b % 2

                @pl.when(j >= 2)
                def _():
                    pltpu.make_async_copy(
                        ones_v, acc.at[cidx[(b - 2) % 4]], ss[b2]).wait()

                @pl.when(j + 1 < cpt)
                def _():
                    pltpu.async_copy(col_hbm.at[base + j + 1],
                                     cidx[(b + 1) % 4], cis[(b + 1) % 4])

                @pl.when(j >= 1)
                def _():
                    pltpu.make_async_copy(col_hbm.at[base + j],
                                          cidx[b], cis[b]).wait()

                pltpu.async_copy(ones_v, acc.at[cidx[b]], ss[b2], add=True)
            return carry

        lax.fori_loop(0, cpt // 4, outer, 0)
        pltpu.make_async_copy(ones_v, acc.at[cidx[(cpt - 2) % 4]], ss[0]).wait()
        pltpu.make_async_copy(ones_v, acc.at[cidx[(cpt - 1) % 4]], ss[1]).wait()
        plsc.subcore_barrier()
        pltpu.sync_copy(acc.at[pl.ds(r0, rpt)], out.at[pl.ds(r0, rpt)])

    return deg_kernel(col2d, ones_blk, zeros128)


def _sc_propagate(g_tab, row2d, col2d, zeros128, n_pad):
    c_pad = row2d.shape[0]
    cpt = c_pad // NS
    rpt = n_pad // NS
    d = g_tab.shape[1]

    @functools.partial(
        pl.kernel,
        out_type=jax.ShapeDtypeStruct((n_pad, d), jnp.float32),
        mesh=_sc_mesh(),
        scratch_types=[
            pltpu.VMEM((CHUNK,), jnp.int32),
            pltpu.VMEM((CHUNK,), jnp.int32),
            pltpu.VMEM((CHUNK,), jnp.int32),
            pltpu.VMEM((CHUNK,), jnp.int32),
            pltpu.VMEM((CHUNK,), jnp.int32),
            pltpu.VMEM((CHUNK,), jnp.int32),
            pltpu.VMEM((CHUNK,), jnp.int32),
            pltpu.VMEM((CHUNK,), jnp.int32),
            pltpu.VMEM((CHUNK, d), jnp.float32),
            pltpu.VMEM((CHUNK, d), jnp.float32),
            pltpu.VMEM_SHARED((n_pad, d), jnp.float32),
            pltpu.SemaphoreType.DMA,
            pltpu.SemaphoreType.DMA,
            pltpu.SemaphoreType.DMA,
            pltpu.SemaphoreType.DMA,
            pltpu.SemaphoreType.DMA,
            pltpu.SemaphoreType.DMA,
            pltpu.SemaphoreType.DMA,
            pltpu.SemaphoreType.DMA,
            pltpu.SemaphoreType.DMA,
            pltpu.SemaphoreType.DMA,
            pltpu.SemaphoreType.DMA,
            pltpu.SemaphoreType.DMA,
        ],
    )
    def prop_kernel(g_hbm, row_hbm, col_hbm, zeros_hbm, out,
                    ri0, ri1, ri2, ri3, ci0, ci1, ci2, ci3, rows0, rows1, acc,
                    rs0, rs1, rs2, rs3, cs0, cs1, cs2, cs3, gs0, gs1, ss0, ss1):
        ridx = (ri0, ri1, ri2, ri3)
        cidx = (ci0, ci1, ci2, ci3)
        rows = (rows0, rows1)
        ris = (rs0, rs1, rs2, rs3)
        cis = (cs0, cs1, cs2, cs3)
        gs = (gs0, gs1)
        ss = (ss0, ss1)
        sid = lax.axis_index("s")
        r0 = sid * rpt
        base = sid * cpt
        pltpu.sync_copy(zeros_hbm.at[pl.ds(r0, rpt)], acc.at[pl.ds(r0, rpt)])
        pltpu.sync_copy(row_hbm.at[base], ridx[0])
        pltpu.sync_copy(col_hbm.at[base], cidx[0])
        pltpu.sync_copy(row_hbm.at[base + 1], ridx[1])
        pltpu.sync_copy(col_hbm.at[base + 1], cidx[1])
        plsc.subcore_barrier()
        pltpu.async_copy(g_hbm.at[ridx[0]], rows[0], gs[0])

        def outer(j4, carry):
            for b in range(4):
                j = j4 * 4 + b
                b2 = b % 2
                ob = 1 - b2

                @pl.when(j >= 1)
                def _():
                    pltpu.make_async_copy(
                        rows[ob], acc.at[cidx[(b - 1) % 4]], ss[ob]).wait()

                @pl.when(j + 1 < cpt)
                def _():
                    @pl.when(j >= 1)
                    def _():
                        pltpu.make_async_copy(
                            row_hbm.at[base + j + 1],
                            ridx[(b + 1) % 4], ris[(b + 1) % 4]).wait()
                    pltpu.async_copy(g_hbm.at[ridx[(b + 1) % 4]], rows[ob], gs[ob])

                @pl.when(j + 2 < cpt)
                def _():
                    pltpu.async_copy(row_hbm.at[base + j + 2],
                                     ridx[(b + 2) % 4], ris[(b + 2) % 4])
                    pltpu.async_copy(col_hbm.at[base + j + 2],
                                     cidx[(b + 2) % 4], cis[(b + 2) % 4])

                pltpu.make_async_copy(g_hbm.at[ridx[b]], rows[b2], gs[b2]).wait()

                @pl.when(j >= 2)
                def _():
                    pltpu.make_async_copy(col_hbm.at[base + j],
                                          cidx[b], cis[b]).wait()

                pltpu.async_copy(rows[b2], acc.at[cidx[b]], ss[b2], add=True)
            return carry

        lax.fori_loop(0, cpt // 4, outer, 0)
        pltpu.make_async_copy(
            rows[(cpt - 1) % 2], acc.at[cidx[(cpt - 1) % 4]],
            ss[(cpt - 1) % 2]).wait()
        plsc.subcore_barrier()
        pltpu.sync_copy(acc.at[pl.ds(r0, rpt)], out.at[pl.ds(r0, rpt)])

    return prop_kernel(g_tab, row2d, col2d, zeros128)


def kernel(x, edge_index, W1, b1, W2, b2, gamma):
    n, d = x.shape
    e = edge_index.shape[1]
    k_steps = gamma.shape[0] - 1

    n_pad = ((n + 2047) // 2048) * 2048
    if n_pad == n:
        n_pad += 2048
    e_unit = NS * 8 * CHUNK
    e_pad = ((e + e_unit - 1) // e_unit) * e_unit

    row = edge_index[0]
    col = edge_index[1]
    if e_pad != e:
        pad = e_pad - e
        row = jnp.concatenate([row, jnp.zeros((pad,), jnp.int32)])
        col = jnp.concatenate([col, jnp.full((pad,), n_pad - 1, jnp.int32)])
    row2d = row.reshape(e_pad // CHUNK, CHUNK)
    col2d = col.reshape(e_pad // CHUNK, CHUNK)

    zeros128 = jnp.zeros((n_pad, d), jnp.float32)
    ones_blk = jnp.ones((CHUNK, d), jnp.float32)

    h = _dense(x, W1, b1, W2, b2)
    h_pad = jnp.pad(h, ((0, n_pad - n), (0, 0)))

    degp = _sc_degree(col2d, ones_blk, zeros128, n_pad)
    dinv, g, out = _prep(degp, h_pad, gamma[0].reshape(1, 1))

    half = row2d.shape[0] // 2
    row_a, row_b = row2d[:half], row2d[half:]
    col_a, col_b = col2d[:half], col2d[half:]
    for k in range(1, k_steps + 1):
        p0 = _sc_propagate(g, row_a, col_a, zeros128, n_pad)
        p1 = _sc_propagate(g, row_b, col_b, zeros128, n_pad)
        out, g = _combine(p0, p1, out, dinv, gamma[k].reshape(1, 1))

    return out[:n]
